# Initial kernel scaffold; baseline (speedup 1.0000x reference)
import jax
import jax.numpy as jnp
from jax.experimental import pallas as pl


def kernel(embeddings, x, gate_w, W1, b1, W2, b2, W3, b3, Ws1, bs1, Ws2, bs2):
    raise NotImplementedError("write your pallas kernel here")


# speedup vs baseline: 1.4512x; 1.4512x over previous
import functools

import jax
import jax.numpy as jnp
from jax.experimental import pallas as pl
from jax.experimental.pallas import tpu as pltpu

EPAD = 128


def _moe_dense_body(
    x32_ref,
    gwt_ref,
    w1_ref,
    w3_ref,
    w2_ref,
    b1_ref,
    b3_ref,
    b2_ref,
    ws1_ref,
    ws2_ref,
    bs1_ref,
    bs2_ref,
    y_ref,
    wdense_ref,
    *, n_experts,
):
    e = pl.program_id(1)
    xb32 = x32_ref[...]
    xb = xb32.astype(jnp.bfloat16)

    @pl.when(e == 0)
    def _gate_and_shared():
        logits = jax.lax.dot_general(
            xb32, gwt_ref[...], (((1,), (0,)), ((), ())),
            preferred_element_type=jnp.float32)
        lane = jax.lax.broadcasted_iota(jnp.int32, logits.shape, 1)
        valid = lane < n_experts
        l = jnp.where(valid, logits, -1e30)
        m = jnp.max(l, axis=1, keepdims=True)
        ex = jnp.where(valid, jnp.exp(l - m), 0.0)
        p = ex / jnp.sum(ex, axis=1, keepdims=True)
        i1 = jnp.argmax(p, axis=1)[:, None]
        oh1 = lane == i1
        i2 = jnp.argmax(jnp.where(oh1, -1.0, p), axis=1)[:, None]
        sel = oh1 | (lane == i2)
        wdense_ref[...] = jnp.where(sel, p, 0.0)

        h1 = jax.lax.dot_general(
            xb, ws1_ref[...], (((1,), (1,)), ((), ())),
            preferred_element_type=jnp.float32) + bs1_ref[...]
        h = (h1 * jax.nn.sigmoid(h1)).astype(jnp.bfloat16)
        z = jax.lax.dot_general(
            h, ws2_ref[...], (((1,), (1,)), ((), ())),
            preferred_element_type=jnp.float32) + bs2_ref[...]
        y_ref[...] = z

    w1 = w1_ref[0]
    w3 = w3_ref[0]
    h1 = jax.lax.dot_general(xb, w1, (((1,), (1,)), ((), ())),
                             preferred_element_type=jnp.float32) + b1_ref[0]
    h3 = jax.lax.dot_general(xb, w3, (((1,), (1,)), ((), ())),
                             preferred_element_type=jnp.float32) + b3_ref[0]
    h = (h1 * jax.nn.sigmoid(h1) * h3).astype(jnp.bfloat16)
    eo = jax.lax.dot_general(h, w2_ref[0], (((1,), (1,)), ((), ())),
                             preferred_element_type=jnp.float32) + b2_ref[0]
    wd = wdense_ref[...]
    lane_e = jax.lax.broadcasted_iota(jnp.int32, wd.shape, 1)
    wcol = jnp.sum(jnp.where(lane_e == e, wd, 0.0), axis=1, keepdims=True)
    y_ref[...] += wcol * eo


def kernel(embeddings, x, gate_w, W1, b1, W2, b2, W3, b3, Ws1, bs1, Ws2, bs2):
    del embeddings
    shape = x.shape
    dim = shape[-1]
    xf = x.reshape(-1, dim)
    t = xf.shape[0]
    n_experts, inter = W1.shape[0], W1.shape[1]

    tb = min(t, 1024)
    grid = (t // tb, n_experts)

    gwt = jnp.zeros((dim, EPAD), jnp.float32).at[:, :n_experts].set(gate_w.T)
    w1b = W1.astype(jnp.bfloat16)
    w3b = W3.astype(jnp.bfloat16)
    w2b = W2.astype(jnp.bfloat16)
    ws1b = Ws1.astype(jnp.bfloat16)
    ws2b = Ws2.astype(jnp.bfloat16)

    y = pl.pallas_call(
        functools.partial(_moe_dense_body, n_experts=n_experts),
        grid=grid,
        in_specs=[
            pl.BlockSpec((tb, dim), lambda i, e: (i, 0)),
            pl.BlockSpec((dim, EPAD), lambda i, e: (0, 0)),
            pl.BlockSpec((1, inter, dim), lambda i, e: (e, 0, 0)),
            pl.BlockSpec((1, inter, dim), lambda i, e: (e, 0, 0)),
            pl.BlockSpec((1, dim, inter), lambda i, e: (e, 0, 0)),
            pl.BlockSpec((1, 1, inter), lambda i, e: (e, 0, 0)),
            pl.BlockSpec((1, 1, inter), lambda i, e: (e, 0, 0)),
            pl.BlockSpec((1, 1, dim), lambda i, e: (e, 0, 0)),
            pl.BlockSpec((inter, dim), lambda i, e: (0, 0)),
            pl.BlockSpec((dim, inter), lambda i, e: (0, 0)),
            pl.BlockSpec((1, inter), lambda i, e: (0, 0)),
            pl.BlockSpec((1, dim), lambda i, e: (0, 0)),
        ],
        out_specs=pl.BlockSpec((tb, dim), lambda i, e: (i, 0)),
        out_shape=jax.ShapeDtypeStruct((t, dim), jnp.float32),
        scratch_shapes=[pltpu.VMEM((tb, EPAD), jnp.float32)],
        compiler_params=pltpu.CompilerParams(
            dimension_semantics=("arbitrary", "arbitrary")),
    )(xf, gwt, w1b, w3b, w2b,
      b1.reshape(n_experts, 1, inter), b3.reshape(n_experts, 1, inter),
      b2.reshape(n_experts, 1, dim),
      ws1b, ws2b, bs1.reshape(1, inter), bs2.reshape(1, dim))
    return y.reshape(shape)
